# baseline (speedup 1.0000x reference)
import functools

import jax
import jax.numpy as jnp
from jax import lax
from jax.experimental import pallas as pl
from jax.experimental.pallas import tpu as pltpu
from jax.experimental.pallas import tpu_sc as plsc

_NBUF = 4


def _emb_kernel_body(B, T, D, NW, idx_hbm, vocab_hbm, pos_hbm, out_hbm,
                     idx_v, pos_v, *bufs_and_sems):
    BB = B // NW
    rows = bufs_and_sems[:_NBUF]
    trows = bufs_and_sems[_NBUF:2 * _NBUF]
    gsems = bufs_and_sems[2 * _NBUF:3 * _NBUF]
    ssems = bufs_and_sems[3 * _NBUF:4 * _NBUF]

    wid = lax.axis_index("s") * 2 + lax.axis_index("c")

    pltpu.sync_copy(idx_hbm.at[:, pl.ds(wid * BB, BB)], idx_v)
    pltpu.sync_copy(pos_hbm.at[pl.ds(0, T)], pos_v)

    def fire_gather(t, b):
        pltpu.async_copy(vocab_hbm.at[idx_v.at[t]], rows[b], gsems[b])

    def wait_gather(b):
        pltpu.make_async_copy(vocab_hbm.at[pl.ds(0, BB)], rows[b], gsems[b]).wait()

    _iota = lax.iota(jnp.int32, 16)
    _ridxs = [_iota + 16 * kb for kb in range(BB // 16)]

    def transpose_add(t, b):
        tv = jnp.full((16,), 0, jnp.int32) + t
        def body(s, carry):
            dp = (_iota + s) & 15
            dpdiv = lax.shift_right_logical(dp, 3)
            dpm7 = lax.shift_left(dp & 7, 7)
            for d0 in range(0, D, 16):
                didx = dp + d0
                dblkv = dpdiv + (d0 // 8)
                pvec = plsc.load_gather(pos_v, [tv, didx])
                for kb in range(BB // 16):
                    ridx = _ridxs[kb]
                    v = plsc.load_gather(rows[b], [ridx, didx])
                    plsc.store_scatter(trows[b], [dblkv, dpm7 + ridx], v + pvec)
            return carry
        lax.fori_loop(0, 16, body, 0)

    def fire_scatter(t, b):
        pltpu.async_copy(trows[b], out_hbm.at[t, :, wid, :], ssems[b])

    def wait_scatter(b):
        pltpu.make_async_copy(trows[b], out_hbm.at[0, :, 0, :], ssems[b]).wait()

    fire_gather(0, 0)
    fire_gather(1, 1)

    def outer(g, carry):
        t0 = _NBUF * g
        for i in range(_NBUF):
            t = t0 + i
            b = i
            b2 = (i + 2) % _NBUF
            wait_gather(b)
            transpose_add(t, b)
            fire_scatter(t, b)
            @pl.when(t + 2 < T)
            def _(t=t, b2=b2):
                @pl.when(t >= 2)
                def _():
                    wait_scatter(b2)
                fire_gather(t + 2, b2)
        return carry

    lax.fori_loop(0, T // _NBUF, outer, 0)

    for t in range(T - 4, T):
        wait_scatter(t % _NBUF)


def kernel(idx, vocab_table, pos_table):
    B, T = idx.shape
    V, D = vocab_table.shape
    NW = 32
    BB = B // NW

    idx_t = jnp.transpose(idx, (1, 0)).astype(jnp.int32)

    mesh = plsc.VectorSubcoreMesh(core_axis_name="c", subcore_axis_name="s",
                                  num_cores=2, num_subcores=16)
    run = functools.partial(
        pl.kernel,
        out_type=jax.ShapeDtypeStruct((T, D // 8, NW, 8 * BB), jnp.float32),
        mesh=mesh,
        scratch_types=[
            pltpu.VMEM((T, BB), jnp.int32),
            pltpu.VMEM((T, D), jnp.float32),
            *[pltpu.VMEM((BB, D), jnp.float32) for _ in range(_NBUF)],
            *[pltpu.VMEM((D // 8, 8 * BB), jnp.float32) for _ in range(_NBUF)],
            *[pltpu.SemaphoreType.DMA for _ in range(2 * _NBUF)],
        ],
        compiler_params=pltpu.CompilerParams(use_tc_tiling_on_sc=False,
                                             needs_layout_passes=False),
    )(functools.partial(_emb_kernel_body, B, T, D, NW))

    out4 = run(idx_t, vocab_table, pos_table)
    out5 = out4.reshape(T, D // 8, NW, 8, BB)
    out = jnp.transpose(out5, (2, 4, 0, 1, 3)).reshape(B, T, D)
    return out


# speedup vs baseline: 1.2424x; 1.2424x over previous
import functools

import jax
import jax.numpy as jnp
from jax import lax
from jax.experimental import pallas as pl
from jax.experimental.pallas import tpu as pltpu
from jax.experimental.pallas import tpu_sc as plsc

_NBUF = 4


def _emb_kernel_body(B, T, D, NW, idx_hbm, vocab_hbm, pos_hbm, out_hbm,
                     idx_v, pos_v, *bufs_and_sems):
    BB = B // NW
    rows = bufs_and_sems[:_NBUF]
    trows = bufs_and_sems[_NBUF:2 * _NBUF]
    gsems = bufs_and_sems[2 * _NBUF:3 * _NBUF]
    ssems = bufs_and_sems[3 * _NBUF:4 * _NBUF]

    wid = lax.axis_index("s") * 2 + lax.axis_index("c")

    pltpu.sync_copy(idx_hbm.at[:, pl.ds(wid * BB, BB)], idx_v)
    pltpu.sync_copy(pos_hbm.at[pl.ds(0, T)], pos_v)

    def fire_gather(t, b):
        pltpu.async_copy(vocab_hbm.at[idx_v.at[t]], rows[b], gsems[b])

    def wait_gather(b):
        pltpu.make_async_copy(vocab_hbm.at[pl.ds(0, BB)], rows[b], gsems[b]).wait()

    _iota = lax.iota(jnp.int32, 16)
    _ridxs = [_iota + 16 * kb for kb in range(BB // 16)]

    def transpose_add(t, b):
        tv = jnp.full((16,), 0, jnp.int32) + t
        @plsc.parallel_loop(0, 16, unroll=2)
        def _(s):
            dp = (_iota + s) & 15
            dpdiv = lax.shift_right_logical(dp, 3)
            dpmod = dp & 7
            for d0 in range(0, D, 16):
                didx = dp + d0
                dblkv = dpdiv + (d0 // 8)
                pvec = plsc.load_gather(pos_v, [tv, didx])
                for kb in range(BB // 16):
                    ridx = _ridxs[kb]
                    v = plsc.load_gather(rows[b], [ridx, didx])
                    plsc.store_scatter(trows[b], [dblkv, dpmod, ridx], v + pvec)

    def fire_scatter(t, b):
        pltpu.async_copy(trows[b], out_hbm.at[t, :, wid], ssems[b])

    def wait_scatter(b):
        pltpu.make_async_copy(trows[b], out_hbm.at[0, :, 0], ssems[b]).wait()

    fire_gather(0, 0)
    fire_gather(1, 1)

    def outer(g, carry):
        t0 = _NBUF * g
        for i in range(_NBUF):
            t = t0 + i
            b = i
            b2 = (i + 2) % _NBUF
            wait_gather(b)
            transpose_add(t, b)
            fire_scatter(t, b)
            @pl.when(t + 2 < T)
            def _(t=t, b2=b2):
                @pl.when(t >= 2)
                def _():
                    wait_scatter(b2)
                fire_gather(t + 2, b2)
        return carry

    lax.fori_loop(0, T // _NBUF, outer, 0)

    for t in range(T - 4, T):
        wait_scatter(t % _NBUF)


def kernel(idx, vocab_table, pos_table):
    B, T = idx.shape
    V, D = vocab_table.shape
    NW = 32
    BB = B // NW

    idx_t = jnp.transpose(idx, (1, 0)).astype(jnp.int32)

    mesh = plsc.VectorSubcoreMesh(core_axis_name="c", subcore_axis_name="s",
                                  num_cores=2, num_subcores=16)
    run = functools.partial(
        pl.kernel,
        out_type=jax.ShapeDtypeStruct((T, D // 8, NW, 8, BB), jnp.float32),
        mesh=mesh,
        scratch_types=[
            pltpu.VMEM((T, BB), jnp.int32),
            pltpu.VMEM((T, D), jnp.float32),
            *[pltpu.VMEM((BB, D), jnp.float32) for _ in range(_NBUF)],
            *[pltpu.VMEM((D // 8, 8, BB), jnp.float32) for _ in range(_NBUF)],
            *[pltpu.SemaphoreType.DMA for _ in range(2 * _NBUF)],
        ],
        compiler_params=pltpu.CompilerParams(use_tc_tiling_on_sc=False,
                                             needs_layout_passes=False),
    )(functools.partial(_emb_kernel_body, B, T, D, NW))

    out5 = run(idx_t, vocab_table, pos_table)
    out = jnp.transpose(out5, (2, 4, 0, 1, 3)).reshape(B, T, D)
    return out


# speedup vs baseline: 1.2504x; 1.0064x over previous
import functools

import jax
import jax.numpy as jnp
from jax import lax
from jax.experimental import pallas as pl
from jax.experimental.pallas import tpu as pltpu
from jax.experimental.pallas import tpu_sc as plsc

_NBUF = 4


def _emb_kernel_body(B, T, D, NW, idx_hbm, vocab_hbm, pos_hbm, out_hbm,
                     idx_v, pos_v, *bufs_and_sems):
    BB = B // NW
    rows = bufs_and_sems[:_NBUF]
    trows = bufs_and_sems[_NBUF:2 * _NBUF]
    gsems = bufs_and_sems[2 * _NBUF:3 * _NBUF]
    ssems = bufs_and_sems[3 * _NBUF:4 * _NBUF]

    wid = lax.axis_index("s") * 2 + lax.axis_index("c")

    pltpu.sync_copy(idx_hbm.at[:, pl.ds(wid * BB, BB)], idx_v)
    pltpu.sync_copy(pos_hbm.at[pl.ds(0, T)], pos_v)

    def fire_gather(t, b):
        pltpu.async_copy(vocab_hbm.at[idx_v.at[t]], rows[b], gsems[b])

    def wait_gather(b):
        pltpu.make_async_copy(vocab_hbm.at[pl.ds(0, BB)], rows[b], gsems[b]).wait()

    _iota = lax.iota(jnp.int32, 16)
    _ridxs = [_iota + 16 * kb for kb in range(BB // 16)]

    def transpose_add(t, b):
        tv = jnp.full((16,), 0, jnp.int32) + t
        @plsc.parallel_loop(0, 16, unroll=4)
        def _(s):
            dp = (_iota + s) & 15
            dpdiv = lax.shift_right_logical(dp, 3)
            dpmod = dp & 7
            for d0 in range(0, D, 16):
                didx = dp + d0
                dblkv = dpdiv + (d0 // 8)
                pvec = plsc.load_gather(pos_v, [tv, didx])
                for kb in range(BB // 16):
                    ridx = _ridxs[kb]
                    v = plsc.load_gather(rows[b], [ridx, didx])
                    plsc.store_scatter(trows[b], [dblkv, dpmod, ridx], v + pvec)

    def fire_scatter(t, b):
        pltpu.async_copy(trows[b], out_hbm.at[t, :, wid], ssems[b])

    def wait_scatter(b):
        pltpu.make_async_copy(trows[b], out_hbm.at[0, :, 0], ssems[b]).wait()

    fire_gather(0, 0)
    fire_gather(1, 1)

    def outer(g, carry):
        t0 = _NBUF * g
        for i in range(_NBUF):
            t = t0 + i
            b = i
            b2 = (i + 2) % _NBUF
            wait_gather(b)
            transpose_add(t, b)
            fire_scatter(t, b)
            @pl.when(t + 2 < T)
            def _(t=t, b2=b2):
                @pl.when(t >= 2)
                def _():
                    wait_scatter(b2)
                fire_gather(t + 2, b2)
        return carry

    lax.fori_loop(0, T // _NBUF, outer, 0)

    for t in range(T - 4, T):
        wait_scatter(t % _NBUF)


def kernel(idx, vocab_table, pos_table):
    B, T = idx.shape
    V, D = vocab_table.shape
    NW = 32
    BB = B // NW

    idx_t = jnp.transpose(idx, (1, 0)).astype(jnp.int32)

    mesh = plsc.VectorSubcoreMesh(core_axis_name="c", subcore_axis_name="s",
                                  num_cores=2, num_subcores=16)
    run = functools.partial(
        pl.kernel,
        out_type=jax.ShapeDtypeStruct((T, D // 8, NW, 8, BB), jnp.float32),
        mesh=mesh,
        scratch_types=[
            pltpu.VMEM((T, BB), jnp.int32),
            pltpu.VMEM((T, D), jnp.float32),
            *[pltpu.VMEM((BB, D), jnp.float32) for _ in range(_NBUF)],
            *[pltpu.VMEM((D // 8, 8, BB), jnp.float32) for _ in range(_NBUF)],
            *[pltpu.SemaphoreType.DMA for _ in range(2 * _NBUF)],
        ],
        compiler_params=pltpu.CompilerParams(use_tc_tiling_on_sc=False,
                                             needs_layout_passes=False),
    )(functools.partial(_emb_kernel_body, B, T, D, NW))

    out5 = run(idx_t, vocab_table, pos_table)
    out = jnp.transpose(out5, (2, 4, 0, 1, 3)).reshape(B, T, D)
    return out
